# Initial kernel scaffold; baseline (speedup 1.0000x reference)
import jax
import jax.numpy as jnp
from jax.experimental import pallas as pl


def kernel(x, edge_index, edge_weight, W1, b1, gamma, beta, Wlin, blin):
    raise NotImplementedError("write your pallas kernel here")


# speedup vs baseline: 17.4271x; 17.4271x over previous
import dataclasses
import functools

import jax
import jax.numpy as jnp
from jax import lax
from jax.experimental import pallas as pl
from jax.experimental.pallas import tpu as pltpu
from jax.experimental.pallas import tpu_sc as plsc

N = 10000
E = 320000
F = 128

NC = 2
NS = 16
NT = NC * NS
EPT = E // NT
RPT = 624
REXTRA = N - NS * RPT
BE = 128
NFULL = EPT // BE
TAIL = EPT - NFULL * BE

_ROW_CHUNKS = ((0, 128), (128, 128), (256, 128), (384, 128), (512, 112))

_MESH = plsc.VectorSubcoreMesh(core_axis_name="c", subcore_axis_name="s")

_SC_PARAMS = pltpu.CompilerParams()
if "needs_layout_passes" in pltpu.CompilerParams.__dataclass_fields__:
    _SC_PARAMS = dataclasses.replace(_SC_PARAMS, needs_layout_passes=False)


@functools.partial(
    pl.kernel,
    mesh=_MESH,
    compiler_params=_SC_PARAMS,
    out_type=jax.ShapeDtypeStruct((NT, 1, N), jnp.float32),
    scratch_types=[
        pltpu.VMEM((EPT,), jnp.int32),
        pltpu.VMEM((EPT,), jnp.float32),
        pltpu.VMEM((N,), jnp.float32),
    ],
)
def _sc_degree(dst_hbm, w_hbm, out_hbm, dst_v, w_v, deg_v):
    c = lax.axis_index("c")
    s = lax.axis_index("s")
    wid = s * NC + c
    base = wid * EPT

    zero16 = jnp.zeros((16,), jnp.float32)

    @pl.loop(0, N, step=16)
    def _(i):
        deg_v[pl.ds(i, 16)] = zero16

    pltpu.sync_copy(dst_hbm.at[pl.ds(base, EPT)], dst_v)
    pltpu.sync_copy(w_hbm.at[pl.ds(base, EPT)], w_v)

    @pl.loop(0, EPT, step=16)
    def _(e):
        idx = dst_v[pl.ds(e, 16)]
        w = w_v[pl.ds(e, 16)]
        plsc.addupdate_scatter(deg_v, [idx], w)

    pltpu.sync_copy(deg_v, out_hbm.at[wid, 0])


@functools.partial(
    pl.kernel,
    mesh=_MESH,
    compiler_params=_SC_PARAMS,
    out_type=jax.ShapeDtypeStruct((NC, N, F), jnp.float32),
    scratch_types=[
        pltpu.VMEM((BE,), jnp.int32),
        pltpu.VMEM((BE,), jnp.int32),
        pltpu.VMEM((BE,), jnp.float32),
        pltpu.VMEM((BE, F), jnp.float32),
        pltpu.VMEM((TAIL,), jnp.int32),
        pltpu.VMEM((TAIL,), jnp.int32),
        pltpu.VMEM((TAIL,), jnp.float32),
        pltpu.VMEM((TAIL, F), jnp.float32),
        pltpu.VMEM_SHARED((N, F), jnp.float32),
    ],
)
def _sc_propagate(g_hbm, src_hbm, dst_hbm, w_hbm, out_hbm,
                  srcv, dstv, wv, rows, srcv2, dstv2, wv2, rows2, acc_sh):
    c = lax.axis_index("c")
    s = lax.axis_index("s")
    wid = s * NC + c
    base = wid * EPT

    zero16 = jnp.zeros((16,), jnp.float32)

    @pl.loop(0, BE)
    def _(r):
        for cc in range(0, F, 16):
            rows[r, pl.ds(cc, 16)] = zero16

    rbase = s * RPT
    for off, sz in _ROW_CHUNKS:
        pltpu.sync_copy(rows.at[pl.ds(0, sz)], acc_sh.at[pl.ds(rbase + off, sz)])

    @pl.when(s == NS - 1)
    def _():
        pltpu.sync_copy(rows.at[pl.ds(0, REXTRA)],
                        acc_sh.at[pl.ds(NS * RPT, REXTRA)])

    plsc.subcore_barrier()

    @pl.loop(0, NFULL)
    def _(b):
        ebase = base + b * BE
        pltpu.sync_copy(src_hbm.at[pl.ds(ebase, BE)], srcv)
        pltpu.sync_copy(dst_hbm.at[pl.ds(ebase, BE)], dstv)
        pltpu.sync_copy(w_hbm.at[pl.ds(ebase, BE)], wv)
        pltpu.sync_copy(g_hbm.at[srcv], rows)

        @pl.loop(0, BE)
        def _(r):
            w_b = plsc.load_gather(wv, [jnp.full((16,), r, jnp.int32)])
            for cc in range(0, F, 16):
                rows[r, pl.ds(cc, 16)] = rows[r, pl.ds(cc, 16)] * w_b

        pltpu.sync_copy(rows, acc_sh.at[dstv], add=True)

    tbase = base + NFULL * BE
    pltpu.sync_copy(src_hbm.at[pl.ds(tbase, TAIL)], srcv2)
    pltpu.sync_copy(dst_hbm.at[pl.ds(tbase, TAIL)], dstv2)
    pltpu.sync_copy(w_hbm.at[pl.ds(tbase, TAIL)], wv2)
    pltpu.sync_copy(g_hbm.at[srcv2], rows2)

    @pl.loop(0, TAIL)
    def _(r):
        w_b = plsc.load_gather(wv2, [jnp.full((16,), r, jnp.int32)])
        for cc in range(0, F, 16):
            rows2[r, pl.ds(cc, 16)] = rows2[r, pl.ds(cc, 16)] * w_b

    pltpu.sync_copy(rows2, acc_sh.at[dstv2], add=True)

    plsc.subcore_barrier()

    for off, sz in _ROW_CHUNKS:
        pltpu.sync_copy(acc_sh.at[pl.ds(rbase + off, sz)],
                        out_hbm.at[c, pl.ds(rbase + off, sz)])

    @pl.when(s == NS - 1)
    def _():
        pltpu.sync_copy(acc_sh.at[pl.ds(NS * RPT, REXTRA)],
                        out_hbm.at[c, pl.ds(NS * RPT, REXTRA)])


def _mm1_body(x_ref, w_ref, o_ref):
    o_ref[...] = jnp.dot(x_ref[...], w_ref[...],
                         preferred_element_type=jnp.float32)


def _scale_body(h_ref, degt_ref, g_ref, dinv_ref):
    deg = jnp.sum(degt_ref[...], axis=1, keepdims=True) + 1.0
    safe = jnp.where(deg > 0, deg, 1.0)
    dinv = jnp.where(deg > 0, lax.rsqrt(safe), 0.0)
    dinv_ref[...] = dinv
    g_ref[...] = h_ref[...] * dinv


def _final_body(accp_ref, g_ref, dinv_ref, b1_ref, gamma_ref, beta_ref,
                wlin_ref, blin_ref, o_ref):
    acc = accp_ref[0] + accp_ref[1] + g_ref[...]
    agg = acc * dinv_ref[...] + b1_ref[...]
    a = jnp.maximum(agg, 0.0)
    mean = jnp.mean(a, axis=0, keepdims=True)
    var = jnp.mean(a * a, axis=0, keepdims=True) - mean * mean
    cscale = gamma_ref[...] * lax.rsqrt(var + 1e-5)
    a_bn = (a - mean) * cscale + beta_ref[...]
    o_ref[...] = jnp.dot(a_bn, wlin_ref[...],
                         preferred_element_type=jnp.float32) + blin_ref[...]


def kernel(x, edge_index, edge_weight, W1, b1, gamma, beta, Wlin, blin):
    src = edge_index[0]
    dst = edge_index[1]

    deg_parts = _sc_degree(dst, edge_weight).reshape(NT, N)
    h = pl.pallas_call(
        _mm1_body,
        out_shape=jax.ShapeDtypeStruct((N, F), jnp.float32),
    )(x, W1)

    g, dinv = pl.pallas_call(
        _scale_body,
        out_shape=[
            jax.ShapeDtypeStruct((N, F), jnp.float32),
            jax.ShapeDtypeStruct((N, 1), jnp.float32),
        ],
    )(h, deg_parts.T)

    acc_parts = _sc_propagate(g, src, dst, edge_weight)

    out = pl.pallas_call(
        _final_body,
        out_shape=jax.ShapeDtypeStruct((N, F), jnp.float32),
    )(acc_parts, g, dinv, b1.reshape(1, F), gamma.reshape(1, F),
      beta.reshape(1, F), Wlin, blin.reshape(1, F))
    return out


# speedup vs baseline: 32.4576x; 1.8625x over previous
import dataclasses
import functools

import jax
import jax.numpy as jnp
from jax import lax
from jax.experimental import pallas as pl
from jax.experimental.pallas import tpu as pltpu
from jax.experimental.pallas import tpu_sc as plsc

N = 10000
E = 320000
F = 128

NC = 2
NS = 16
NT = NC * NS
EPT = E // NT
RPT = 624
REXTRA = N - NS * RPT
BE = 80
NBLK = E // BE
NB0 = NBLK // NT
EALL = NB0 * BE

_ROW_CHUNKS = ((0, 128), (128, 128), (256, 128), (384, 128), (512, 112))

_MESH = plsc.VectorSubcoreMesh(core_axis_name="c", subcore_axis_name="s")

_SC_PARAMS = pltpu.CompilerParams()
if "needs_layout_passes" in pltpu.CompilerParams.__dataclass_fields__:
    _SC_PARAMS = dataclasses.replace(_SC_PARAMS, needs_layout_passes=False)


@functools.partial(
    pl.kernel,
    mesh=_MESH,
    compiler_params=_SC_PARAMS,
    out_type=jax.ShapeDtypeStruct((NT, 1, N), jnp.float32),
    scratch_types=[
        pltpu.VMEM((EPT,), jnp.int32),
        pltpu.VMEM((EPT,), jnp.float32),
        pltpu.VMEM((N,), jnp.float32),
    ],
)
def _sc_degree(dst_hbm, w_hbm, out_hbm, dst_v, w_v, deg_v):
    c = lax.axis_index("c")
    s = lax.axis_index("s")
    wid = s * NC + c
    base = wid * EPT

    zero16 = jnp.zeros((16,), jnp.float32)

    @pl.loop(0, N, step=16)
    def _(i):
        deg_v[pl.ds(i, 16)] = zero16

    pltpu.sync_copy(dst_hbm.at[pl.ds(base, EPT)], dst_v)
    pltpu.sync_copy(w_hbm.at[pl.ds(base, EPT)], w_v)

    @pl.loop(0, EPT, step=16)
    def _(e):
        idx = dst_v[pl.ds(e, 16)]
        w = w_v[pl.ds(e, 16)]
        plsc.addupdate_scatter(deg_v, [idx], w)

    pltpu.sync_copy(deg_v, out_hbm.at[wid, 0])


@functools.partial(
    pl.kernel,
    mesh=_MESH,
    compiler_params=_SC_PARAMS,
    out_type=jax.ShapeDtypeStruct((NC, N, F), jnp.float32),
    scratch_types=[
        pltpu.VMEM((EALL,), jnp.int32),
        pltpu.VMEM((EALL,), jnp.int32),
        pltpu.VMEM((EALL,), jnp.float32),
        pltpu.VMEM((BE, F), jnp.float32),
        pltpu.VMEM((BE, F), jnp.float32),
        pltpu.VMEM_SHARED((N, F), jnp.float32),
        pltpu.SemaphoreType.DMA,
        pltpu.SemaphoreType.DMA,
        pltpu.SemaphoreType.DMA,
        pltpu.SemaphoreType.DMA,
        pltpu.SemaphoreType.DMA,
    ],
)
def _sc_propagate(g_hbm, src_hbm, dst_hbm, w_hbm, out_hbm,
                  src_all, dst_all, w_all, rows0, rows1, acc_sh,
                  sg0, sg1, ss0, ss1, sst):
    c = lax.axis_index("c")
    s = lax.axis_index("s")
    wid = s * NC + c
    blk_base = wid * NB0
    ebase = blk_base * BE

    st1 = pltpu.make_async_copy(src_hbm.at[pl.ds(ebase, EALL)], src_all, sst)
    st2 = pltpu.make_async_copy(w_hbm.at[pl.ds(ebase, EALL)], w_all, sst)
    st3 = pltpu.make_async_copy(dst_hbm.at[pl.ds(ebase, EALL)], dst_all, sst)
    st1.start()
    st2.start()
    st3.start()

    zero16 = jnp.zeros((16,), jnp.float32)

    @pl.loop(0, BE)
    def _(r):
        for cc in range(0, F, 16):
            rows0[r, pl.ds(cc, 16)] = zero16

    rbase = s * RPT
    for off in range(0, RPT - BE + 1, BE):
        pltpu.sync_copy(rows0, acc_sh.at[pl.ds(rbase + off, BE)])
    _zrem = RPT % BE
    pltpu.sync_copy(rows0.at[pl.ds(0, _zrem)],
                    acc_sh.at[pl.ds(rbase + RPT - _zrem, _zrem)])

    @pl.when(s == NS - 1)
    def _():
        pltpu.sync_copy(rows0.at[pl.ds(0, REXTRA)],
                        acc_sh.at[pl.ds(NS * RPT, REXTRA)])

    st1.wait()
    st2.wait()
    st3.wait()

    plsc.subcore_barrier()

    def gather(b, rows_ref, sem):
        return pltpu.make_async_copy(
            g_hbm.at[src_all.at[pl.ds(b * BE, BE)]], rows_ref, sem)

    def _scatter_descs(b, rows_ref, sem):
        for gph in range(BE // 16):
            idx = dst_all[pl.ds(b * BE + gph * 16, 16)]
            yield pltpu.make_async_copy(rows_ref.at[pl.ds(gph * 16, 16)],
                                        acc_sh.at[idx], sem)

    def scatter_start(b, rows_ref, sem):
        for d in _scatter_descs(b, rows_ref, sem):
            d.start(add=True)

    def scatter_wait(b, rows_ref, sem):
        for d in _scatter_descs(b, rows_ref, sem):
            d.wait()

    def scale(b, rows_ref):
        @pl.loop(0, BE)
        def _(r):
            w_b = plsc.load_gather(w_all, [jnp.full((16,), b * BE + r,
                                                    jnp.int32)])
            for cc in range(0, F, 16):
                rows_ref[r, pl.ds(cc, 16)] = rows_ref[r, pl.ds(cc, 16)] * w_b

    gather(0, rows0, sg0).start()
    gather(1, rows1, sg1).start()
    gather(0, rows0, sg0).wait()
    scale(0, rows0)
    scatter_start(0, rows0, ss0)

    @pl.loop(1, NB0 - 2, step=2)
    def _(b):
        gather(b, rows1, sg1).wait()
        scatter_wait(b - 1, rows0, ss0)
        gather(b + 1, rows0, sg0).start()
        scale(b, rows1)
        scatter_start(b, rows1, ss1)
        gather(b + 1, rows0, sg0).wait()
        scatter_wait(b, rows1, ss1)
        gather(b + 2, rows1, sg1).start()
        scale(b + 1, rows0)
        scatter_start(b + 1, rows0, ss0)

    gather(NB0 - 2, rows1, sg1).wait()
    scatter_wait(NB0 - 3, rows0, ss0)
    gather(NB0 - 1, rows0, sg0).start()
    scale(NB0 - 2, rows1)
    scatter_start(NB0 - 2, rows1, ss1)

    gather(NB0 - 1, rows0, sg0).wait()
    scatter_wait(NB0 - 2, rows1, ss1)
    scale(NB0 - 1, rows0)
    scatter_start(NB0 - 1, rows0, ss0)
    scatter_wait(NB0 - 1, rows0, ss0)

    plsc.subcore_barrier()

    for off, sz in _ROW_CHUNKS:
        pltpu.sync_copy(acc_sh.at[pl.ds(rbase + off, sz)],
                        out_hbm.at[c, pl.ds(rbase + off, sz)])

    @pl.when(s == NS - 1)
    def _():
        pltpu.sync_copy(acc_sh.at[pl.ds(NS * RPT, REXTRA)],
                        out_hbm.at[c, pl.ds(NS * RPT, REXTRA)])


def _mm1_body(x_ref, w_ref, o_ref):
    o_ref[...] = jnp.dot(x_ref[...], w_ref[...],
                         preferred_element_type=jnp.float32)


def _scale_body(h_ref, degt_ref, g_ref, dinv_ref):
    deg = jnp.sum(degt_ref[...], axis=1, keepdims=True) + 1.0
    safe = jnp.where(deg > 0, deg, 1.0)
    dinv = jnp.where(deg > 0, lax.rsqrt(safe), 0.0)
    dinv_ref[...] = dinv
    g_ref[...] = h_ref[...] * dinv


def _final_body(accp_ref, g_ref, dinv_ref, b1_ref, gamma_ref, beta_ref,
                wlin_ref, blin_ref, o_ref):
    acc = accp_ref[0] + accp_ref[1] + g_ref[...]
    agg = acc * dinv_ref[...] + b1_ref[...]
    a = jnp.maximum(agg, 0.0)
    mean = jnp.mean(a, axis=0, keepdims=True)
    var = jnp.mean(a * a, axis=0, keepdims=True) - mean * mean
    cscale = gamma_ref[...] * lax.rsqrt(var + 1e-5)
    a_bn = (a - mean) * cscale + beta_ref[...]
    o_ref[...] = jnp.dot(a_bn, wlin_ref[...],
                         preferred_element_type=jnp.float32) + blin_ref[...]


def kernel(x, edge_index, edge_weight, W1, b1, gamma, beta, Wlin, blin):
    src = edge_index[0]
    dst = edge_index[1]

    deg_parts = _sc_degree(dst, edge_weight).reshape(NT, N)
    h = pl.pallas_call(
        _mm1_body,
        out_shape=jax.ShapeDtypeStruct((N, F), jnp.float32),
    )(x, W1)

    g, dinv = pl.pallas_call(
        _scale_body,
        out_shape=[
            jax.ShapeDtypeStruct((N, F), jnp.float32),
            jax.ShapeDtypeStruct((N, 1), jnp.float32),
        ],
    )(h, deg_parts.T)

    acc_parts = _sc_propagate(g, src, dst, edge_weight)

    out = pl.pallas_call(
        _final_body,
        out_shape=jax.ShapeDtypeStruct((N, F), jnp.float32),
    )(acc_parts, g, dinv, b1.reshape(1, F), gamma.reshape(1, F),
      beta.reshape(1, F), Wlin, blin.reshape(1, F))
    return out
